# baseline (speedup 1.0000x reference)
import functools

import jax
import jax.numpy as jnp
from jax import lax
from jax.experimental import pallas as pl
from jax.experimental.pallas import tpu as pltpu
from jax.experimental.pallas import tpu_sc as plsc

N = 10000
E = 320000
D = 128
H = 128
G = 64
EPS = 1e-5

TILES = 32
NPT = 313
NPAD = TILES * NPT
CH = 4000
SB = 128
MB = CH + SB
DUMP = MB - 16
NEG = -3.0e38


def _sc_agg_body(with_attr, *refs):
    if with_attr:
        (dst_hbm, src_hbm, feats_hbm, attr_hbm, aggx_hbm, aggattr_hbm,
         dstbuf, srcbuf, mldst, msrc, meid, rows, attrrows, accx, accattr,
         sem, sem2) = refs
    else:
        (dst_hbm, src_hbm, feats_hbm, aggx_hbm,
         dstbuf, srcbuf, mldst, msrc, rows, accx,
         sem) = refs

    cid = lax.axis_index("c")
    sid = lax.axis_index("s")
    wid = sid * 2 + cid
    lo = wid * NPT
    hi = lo + NPT
    lane = lax.iota(jnp.int32, 16)
    neg16 = jnp.full((16,), NEG, jnp.float32)
    zero16 = jnp.zeros((16,), jnp.int32)
    one16 = jnp.full((16,), 1, jnp.int32)
    dump16 = jnp.full((16,), DUMP, jnp.int32)

    def init_accx(i, c):
        accx[pl.ds(16 * i, 16)] = neg16
        return c
    lax.fori_loop(0, NPT * D // 16, init_accx, 0)
    if with_attr:
        def init_accattr(i, c):
            accattr[pl.ds(16 * i, 16)] = neg16
            return c
        lax.fori_loop(0, NPT, init_accattr, 0)

    def init_midx(i, c):
        msrc[pl.ds(16 * i, 16)] = zero16
        if with_attr:
            meid[pl.ds(16 * i, 16)] = zero16
        return c
    lax.fori_loop(0, MB // 16, init_midx, 0)

    def chunk_body(c, carry):
        base = c * CH
        pltpu.sync_copy(dst_hbm.at[pl.ds(base, CH)], dstbuf)
        pltpu.sync_copy(src_hbm.at[pl.ds(base, CH)], srcbuf)

        def scan_g(g, pos):
            v = dstbuf[pl.ds(16 * g, 16)]
            m = (v >= lo) & (v < hi)
            mi = m.astype(jnp.int32)
            s = plsc.cumsum(mi)
            idx = jnp.where(m, pos + (s - mi), DUMP)
            plsc.store_scatter(mldst, [idx], v - lo)
            plsc.store_scatter(msrc, [idx], srcbuf[pl.ds(16 * g, 16)])
            if with_attr:
                plsc.store_scatter(meid, [idx], lane + (base + 16 * g))
            return pos + jnp.max(s)
        nmatch = lax.fori_loop(0, CH // 16, scan_g, 0)

        def sb_body(sb, carry2):
            off = sb * SB
            cnt = jnp.minimum(nmatch - off, SB)
            pltpu.async_copy(feats_hbm.at[msrc.at[pl.ds(off, SB)]],
                             rows, sem).wait()
            if with_attr:
                pltpu.async_copy(attr_hbm.at[meid.at[pl.ds(off, SB)]],
                                 attrrows, sem2).wait()

            def edge_body(i, carry3):
                d = mldst[pl.ds(off + i, 16)][0]
                ab = d * D
                for j in range(D // 16):
                    sl = pl.ds(ab + 16 * j, 16)
                    accx[sl] = jnp.maximum(accx[sl], rows[i, pl.ds(16 * j, 16)])
                if with_attr:
                    sa = pl.ds(d * 16, 16)
                    accattr[sa] = jnp.maximum(accattr[sa],
                                              attrrows[i, pl.ds(0, 16)])
                return carry3
            lax.fori_loop(0, cnt, edge_body, 0)
            return carry2
        lax.fori_loop(0, (nmatch + SB - 1) // SB, sb_body, 0)
        return carry
    lax.fori_loop(0, E // CH, chunk_body, 0)

    pltpu.sync_copy(accx, aggx_hbm.at[pl.ds(lo * D, NPT * D)])
    if with_attr:
        pltpu.sync_copy(accattr, aggattr_hbm.at[pl.ds(lo * 16, NPT * 16)])


def _sc_aggregate(feats, src, dst, attr16, with_attr):
    mesh = plsc.VectorSubcoreMesh(core_axis_name="c", subcore_axis_name="s")
    if with_attr:
        out_type = [jax.ShapeDtypeStruct((NPAD * D,), jnp.float32),
                    jax.ShapeDtypeStruct((NPAD * 16,), jnp.float32)]
        scratch = [
            pltpu.VMEM((CH,), jnp.int32),
            pltpu.VMEM((CH,), jnp.int32),
            pltpu.VMEM((MB,), jnp.int32),
            pltpu.VMEM((MB,), jnp.int32),
            pltpu.VMEM((MB,), jnp.int32),
            pltpu.VMEM((SB, D), jnp.float32),
            pltpu.VMEM((SB, 16), jnp.float32),
            pltpu.VMEM((NPT * D,), jnp.float32),
            pltpu.VMEM((NPT * 16,), jnp.float32),
            pltpu.SemaphoreType.DMA,
            pltpu.SemaphoreType.DMA,
        ]
        f = pl.kernel(functools.partial(_sc_agg_body, True),
                      out_type=out_type, mesh=mesh, scratch_types=scratch,
                      compiler_params=pltpu.CompilerParams(
                          needs_layout_passes=False,
                          use_tc_tiling_on_sc=False))
        return f(dst, src, feats, attr16)
    out_type = jax.ShapeDtypeStruct((NPAD * D,), jnp.float32)
    scratch = [
        pltpu.VMEM((CH,), jnp.int32),
        pltpu.VMEM((CH,), jnp.int32),
        pltpu.VMEM((MB,), jnp.int32),
        pltpu.VMEM((MB,), jnp.int32),
        pltpu.VMEM((SB, D), jnp.float32),
        pltpu.VMEM((NPT * D,), jnp.float32),
        pltpu.SemaphoreType.DMA,
    ]
    f = pl.kernel(functools.partial(_sc_agg_body, False),
                  out_type=out_type, mesh=mesh, scratch_types=scratch,
                  compiler_params=pltpu.CompilerParams(
                      needs_layout_passes=False,
                      use_tc_tiling_on_sc=False))
    return f(dst, src, feats)


def _fix(a):
    return jnp.where(jnp.isfinite(a) & (a > NEG), a, 0.0)


def _dense1_body(ax_ref, aa_ref, wx_ref, wa_ref, b_ref, g_ref, bt_ref, o_ref):
    ax = _fix(ax_ref[...])
    aa = _fix(aa_ref[...])
    h = (jnp.dot(ax, wx_ref[...], preferred_element_type=jnp.float32)
         + jnp.dot(aa, wa_ref[...], preferred_element_type=jnp.float32)
         + b_ref[...])
    row = lax.broadcasted_iota(jnp.int32, (NPAD, 1), 0)
    mask = (row < N).astype(jnp.float32)
    mean = jnp.sum(h * mask, axis=0, keepdims=True) * (1.0 / N)
    d = (h - mean) * mask
    var = jnp.sum(d * d, axis=0, keepdims=True) * (1.0 / N)
    hn = (h - mean) * lax.rsqrt(var + EPS) * g_ref[...] + bt_ref[...]
    o_ref[...] = jnp.maximum(hn, 0.0) * mask


def _dense2_body(ax_ref, aa_ref, wx_ref, wa_ref, b_ref, g_ref, bt_ref,
                 wout_ref, bout_ref, batch_ref, o_ref):
    ax = _fix(ax_ref[...])
    aa = _fix(aa_ref[...])
    h = (jnp.dot(ax, wx_ref[...], preferred_element_type=jnp.float32)
         + jnp.dot(aa, wa_ref[...], preferred_element_type=jnp.float32)
         + b_ref[...])
    row = lax.broadcasted_iota(jnp.int32, (NPAD, 1), 0)
    mask = (row < N).astype(jnp.float32)
    mean = jnp.sum(h * mask, axis=0, keepdims=True) * (1.0 / N)
    d = (h - mean) * mask
    var = jnp.sum(d * d, axis=0, keepdims=True) * (1.0 / N)
    hn = (h - mean) * lax.rsqrt(var + EPS) * g_ref[...] + bt_ref[...]
    h1 = jnp.maximum(hn, 0.0)
    valid = row < N
    b = batch_ref[...]
    cols = []
    for gidx in range(G):
        sel = jnp.where((b == gidx) & valid, h1, -jnp.inf)
        cols.append(jnp.max(sel, axis=0, keepdims=True))
    pooled = jnp.concatenate(cols, axis=0)
    pooled = jnp.where(jnp.isfinite(pooled), pooled, 0.0)
    o_ref[...] = (jnp.dot(pooled, wout_ref[...],
                          preferred_element_type=jnp.float32) + bout_ref[...])


def kernel(x, edge_index, edge_attr, batch, W0, b0, g0, bt0, W1, b1, g1, bt1,
           Wout, bout):
    src = edge_index[0]
    dst = edge_index[1]
    attr16 = jnp.pad(edge_attr, ((0, 0), (0, 10)))
    xpad = jnp.pad(x, ((0, NPAD - N), (0, 0)))
    batch2d = jnp.pad(batch, (0, NPAD - N), constant_values=G).reshape(NPAD, 1)

    aggx0_f, aggattr_f = _sc_aggregate(xpad, src, dst, attr16, True)
    aggx0 = aggx0_f.reshape(NPAD, D)
    aggattr = aggattr_f.reshape(NPAD, 16)

    W0x = W0[:D]
    W0a = jnp.pad(W0[D:], ((0, 10), (0, 0)))
    h0 = pl.pallas_call(
        _dense1_body,
        out_shape=jax.ShapeDtypeStruct((NPAD, H), jnp.float32),
    )(aggx0, aggattr, W0x, W0a, b0.reshape(1, H), g0.reshape(1, H),
      bt0.reshape(1, H))

    aggx1 = _sc_aggregate(h0, src, dst, None, False).reshape(NPAD, D)

    W1x = W1[:H]
    W1a = jnp.pad(W1[H:], ((0, 10), (0, 0)))
    out = pl.pallas_call(
        _dense2_body,
        out_shape=jax.ShapeDtypeStruct((G, 1), jnp.float32),
    )(aggx1, aggattr, W1x, W1a, b1.reshape(1, H), g1.reshape(1, H),
      bt1.reshape(1, H), Wout, bout.reshape(1, 1), batch2d)
    return out


# speedup vs baseline: 1.8660x; 1.8660x over previous
import functools

import jax
import jax.numpy as jnp
from jax import lax
from jax.experimental import pallas as pl
from jax.experimental.pallas import tpu as pltpu
from jax.experimental.pallas import tpu_sc as plsc

N = 10000
E = 320000
D = 128
H = 128
G = 64
EPS = 1e-5

TILES = 32
NPT = 313
NPAD = TILES * NPT
CH = 6400
NCHUNK = E // CH
SB = 128
MB = CH + SB
DUMP = MB - 16
NEG = -3.0e38


def _sc_agg_body(with_attr, *refs):
    if with_attr:
        (edges_hbm, feats_hbm, attr_hbm, aggx_hbm, aggattr_hbm,
         ebuf0, ebuf1, mldst, msrc, meid, rows0, rows1, arows0, arows1,
         accx, accattr, esem0, esem1, rsem0, rsem1, asem0, asem1) = refs
    else:
        (edges_hbm, feats_hbm, aggx_hbm,
         ebuf0, ebuf1, mldst, msrc, rows0, rows1,
         accx, esem0, esem1, rsem0, rsem1) = refs
    ebuf = (ebuf0, ebuf1)
    esem = (esem0, esem1)
    rowsb = (rows0, rows1)
    rsem = (rsem0, rsem1)
    if with_attr:
        arowsb = (arows0, arows1)
        asem = (asem0, asem1)

    cid = lax.axis_index("c")
    sid = lax.axis_index("s")
    wid = sid * 2 + cid
    lo = wid * NPT
    hi = lo + NPT
    lane = lax.iota(jnp.int32, 16)
    neg16 = jnp.full((16,), NEG, jnp.float32)
    zero16 = jnp.zeros((16,), jnp.int32)

    def init_accx(i, c):
        accx[pl.ds(16 * i, 16)] = neg16
        return c
    lax.fori_loop(0, NPT * D // 16, init_accx, 0)
    if with_attr:
        def init_accattr(i, c):
            accattr[pl.ds(16 * i, 16)] = neg16
            return c
        lax.fori_loop(0, NPT, init_accattr, 0)

    def init_midx(i, c):
        msrc[pl.ds(16 * i, 16)] = zero16
        if with_attr:
            meid[pl.ds(16 * i, 16)] = zero16
        return c
    lax.fori_loop(0, MB // 16, init_midx, 0)

    pltpu.async_copy(edges_hbm.at[pl.ds(0, 2)], ebuf[0], esem[0])

    def do_chunk(c, p):
        base = c * CH
        @pl.when(c + 1 < NCHUNK)
        def _():
            pltpu.async_copy(edges_hbm.at[pl.ds(2 * (c + 1), 2)],
                             ebuf[1 - p], esem[1 - p])
        pltpu.make_async_copy(edges_hbm.at[pl.ds(2 * c, 2)],
                              ebuf[p], esem[p]).wait()
        eb = ebuf[p]

        def scan_g(g, pos):
            v = eb[0, pl.ds(16 * g, 16)]
            m = (v >= lo) & (v < hi)
            mi = m.astype(jnp.int32)
            s = plsc.cumsum(mi)
            idx = jnp.where(m, pos + (s - mi), DUMP)
            plsc.store_scatter(mldst, [idx], v - lo)
            plsc.store_scatter(msrc, [idx], eb[1, pl.ds(16 * g, 16)])
            if with_attr:
                plsc.store_scatter(meid, [idx], lane + (base + 16 * g))
            return pos + jnp.max(s)
        nmatch = lax.fori_loop(0, CH // 16, scan_g, 0)
        nsb = (nmatch + SB - 1) // SB

        def issue(sb, q):
            off = sb * SB
            pltpu.async_copy(feats_hbm.at[msrc.at[pl.ds(off, SB)]],
                             rowsb[q], rsem[q])
            if with_attr:
                pltpu.async_copy(attr_hbm.at[meid.at[pl.ds(off, SB)]],
                                 arowsb[q], asem[q])

        def consume(sb, q):
            off = sb * SB
            cnt = jnp.minimum(nmatch - off, SB)
            pltpu.make_async_copy(feats_hbm.at[msrc.at[pl.ds(off, SB)]],
                                  rowsb[q], rsem[q]).wait()
            if with_attr:
                pltpu.make_async_copy(attr_hbm.at[meid.at[pl.ds(off, SB)]],
                                      arowsb[q], asem[q]).wait()

            def edge_body(i, carry3):
                d = mldst[pl.ds(off + i, 16)][0]
                ab = d * D
                for j in range(D // 16):
                    sl = pl.ds(ab + 16 * j, 16)
                    accx[sl] = jnp.maximum(accx[sl],
                                           rowsb[q][i, pl.ds(16 * j, 16)])
                if with_attr:
                    sa = pl.ds(d * 16, 16)
                    accattr[sa] = jnp.maximum(accattr[sa],
                                              arowsb[q][i, pl.ds(0, 16)])
                return carry3
            lax.fori_loop(0, cnt, edge_body, 0)

        @pl.when(nsb > 0)
        def _():
            issue(0, 0)

        def sb_pair(h, carry2):
            for q in range(2):
                sb = 2 * h + q

                @pl.when(sb < nsb)
                def _():
                    @pl.when(sb + 1 < nsb)
                    def _():
                        issue(sb + 1, 1 - q)
                    consume(sb, q)
            return carry2
        lax.fori_loop(0, (nsb + 1) // 2, sb_pair, 0)
        return nmatch

    def chunk_pair(hc, carry):
        for p in range(2):
            c = 2 * hc + p
            if NCHUNK % 2 != 0:
                raise ValueError("NCHUNK must be even")
            do_chunk(c, p)
        return carry
    lax.fori_loop(0, NCHUNK // 2, chunk_pair, 0)

    pltpu.sync_copy(accx, aggx_hbm.at[pl.ds(lo * D, NPT * D)])
    if with_attr:
        pltpu.sync_copy(accattr, aggattr_hbm.at[pl.ds(lo * 16, NPT * 16)])


def _sc_aggregate(feats, edges, attr16, with_attr):
    mesh = plsc.VectorSubcoreMesh(core_axis_name="c", subcore_axis_name="s")
    params = pltpu.CompilerParams(needs_layout_passes=False,
                                  use_tc_tiling_on_sc=False)
    if with_attr:
        out_type = [jax.ShapeDtypeStruct((NPAD * D,), jnp.float32),
                    jax.ShapeDtypeStruct((NPAD * 16,), jnp.float32)]
        scratch = [
            pltpu.VMEM((2, CH), jnp.int32),
            pltpu.VMEM((2, CH), jnp.int32),
            pltpu.VMEM((MB,), jnp.int32),
            pltpu.VMEM((MB,), jnp.int32),
            pltpu.VMEM((MB,), jnp.int32),
            pltpu.VMEM((SB, D), jnp.float32),
            pltpu.VMEM((SB, D), jnp.float32),
            pltpu.VMEM((SB, 16), jnp.float32),
            pltpu.VMEM((SB, 16), jnp.float32),
            pltpu.VMEM((NPT * D,), jnp.float32),
            pltpu.VMEM((NPT * 16,), jnp.float32),
            pltpu.SemaphoreType.DMA,
            pltpu.SemaphoreType.DMA,
            pltpu.SemaphoreType.DMA,
            pltpu.SemaphoreType.DMA,
            pltpu.SemaphoreType.DMA,
            pltpu.SemaphoreType.DMA,
        ]
        f = pl.kernel(functools.partial(_sc_agg_body, True),
                      out_type=out_type, mesh=mesh, scratch_types=scratch,
                      compiler_params=params)
        return f(edges, feats, attr16)
    out_type = jax.ShapeDtypeStruct((NPAD * D,), jnp.float32)
    scratch = [
        pltpu.VMEM((2, CH), jnp.int32),
        pltpu.VMEM((2, CH), jnp.int32),
        pltpu.VMEM((MB,), jnp.int32),
        pltpu.VMEM((MB,), jnp.int32),
        pltpu.VMEM((SB, D), jnp.float32),
        pltpu.VMEM((SB, D), jnp.float32),
        pltpu.VMEM((NPT * D,), jnp.float32),
        pltpu.SemaphoreType.DMA,
        pltpu.SemaphoreType.DMA,
        pltpu.SemaphoreType.DMA,
        pltpu.SemaphoreType.DMA,
    ]
    f = pl.kernel(functools.partial(_sc_agg_body, False),
                  out_type=out_type, mesh=mesh, scratch_types=scratch,
                  compiler_params=params)
    return f(edges, feats)


def _fix(a):
    return jnp.where(jnp.isfinite(a) & (a > NEG), a, 0.0)


def _dense1_body(ax_ref, aa_ref, wx_ref, wa_ref, b_ref, g_ref, bt_ref, o_ref):
    ax = _fix(ax_ref[...])
    aa = _fix(aa_ref[...])
    h = (jnp.dot(ax, wx_ref[...], preferred_element_type=jnp.float32)
         + jnp.dot(aa, wa_ref[...], preferred_element_type=jnp.float32)
         + b_ref[...])
    row = lax.broadcasted_iota(jnp.int32, (NPAD, 1), 0)
    mask = (row < N).astype(jnp.float32)
    mean = jnp.sum(h * mask, axis=0, keepdims=True) * (1.0 / N)
    d = (h - mean) * mask
    var = jnp.sum(d * d, axis=0, keepdims=True) * (1.0 / N)
    hn = (h - mean) * lax.rsqrt(var + EPS) * g_ref[...] + bt_ref[...]
    o_ref[...] = jnp.maximum(hn, 0.0) * mask


def _dense2_body(ax_ref, aa_ref, wx_ref, wa_ref, b_ref, g_ref, bt_ref,
                 wout_ref, bout_ref, batch_ref, o_ref):
    ax = _fix(ax_ref[...])
    aa = _fix(aa_ref[...])
    h = (jnp.dot(ax, wx_ref[...], preferred_element_type=jnp.float32)
         + jnp.dot(aa, wa_ref[...], preferred_element_type=jnp.float32)
         + b_ref[...])
    row = lax.broadcasted_iota(jnp.int32, (NPAD, 1), 0)
    mask = (row < N).astype(jnp.float32)
    mean = jnp.sum(h * mask, axis=0, keepdims=True) * (1.0 / N)
    d = (h - mean) * mask
    var = jnp.sum(d * d, axis=0, keepdims=True) * (1.0 / N)
    hn = (h - mean) * lax.rsqrt(var + EPS) * g_ref[...] + bt_ref[...]
    h1 = jnp.maximum(hn, 0.0)
    valid = row < N
    b = batch_ref[...]
    cols = []
    for gidx in range(G):
        sel = jnp.where((b == gidx) & valid, h1, -jnp.inf)
        cols.append(jnp.max(sel, axis=0, keepdims=True))
    pooled = jnp.concatenate(cols, axis=0)
    pooled = jnp.where(jnp.isfinite(pooled), pooled, 0.0)
    o_ref[...] = (jnp.dot(pooled, wout_ref[...],
                          preferred_element_type=jnp.float32) + bout_ref[...])


def kernel(x, edge_index, edge_attr, batch, W0, b0, g0, bt0, W1, b1, g1, bt1,
           Wout, bout):
    dst = edge_index[1].reshape(NCHUNK, 1, CH)
    src = edge_index[0].reshape(NCHUNK, 1, CH)
    edges = jnp.concatenate([dst, src], axis=1).reshape(2 * NCHUNK, CH)
    attr16 = jnp.pad(edge_attr, ((0, 0), (0, 10)))
    xpad = jnp.pad(x, ((0, NPAD - N), (0, 0)))
    batch2d = jnp.pad(batch, (0, NPAD - N), constant_values=G).reshape(NPAD, 1)

    aggx0_f, aggattr_f = _sc_aggregate(xpad, edges, attr16, True)
    aggx0 = aggx0_f.reshape(NPAD, D)
    aggattr = aggattr_f.reshape(NPAD, 16)

    W0x = W0[:D]
    W0a = jnp.pad(W0[D:], ((0, 10), (0, 0)))
    h0 = pl.pallas_call(
        _dense1_body,
        out_shape=jax.ShapeDtypeStruct((NPAD, H), jnp.float32),
    )(aggx0, aggattr, W0x, W0a, b0.reshape(1, H), g0.reshape(1, H),
      bt0.reshape(1, H))

    aggx1 = _sc_aggregate(h0, edges, None, False).reshape(NPAD, D)

    W1x = W1[:H]
    W1a = jnp.pad(W1[H:], ((0, 10), (0, 0)))
    out = pl.pallas_call(
        _dense2_body,
        out_shape=jax.ShapeDtypeStruct((G, 1), jnp.float32),
    )(aggx1, aggattr, W1x, W1a, b1.reshape(1, H), g1.reshape(1, H),
      bt1.reshape(1, H), Wout, bout.reshape(1, 1), batch2d)
    return out


# speedup vs baseline: 1.9719x; 1.0568x over previous
import functools

import jax
import jax.numpy as jnp
from jax import lax
from jax.experimental import pallas as pl
from jax.experimental.pallas import tpu as pltpu
from jax.experimental.pallas import tpu_sc as plsc

N = 10000
E = 320000
D = 128
H = 128
G = 64
EPS = 1e-5

TILES = 32
NPT = 313
NPAD = TILES * NPT
CH = 6400
NCHUNK = E // CH
SB = 128
MB = CH + SB
DUMP = MB - 16
REG = E + 2 * CH
NEG = -3.0e38


def _sc_l1_body(edges_hbm, feats_hbm, attr_hbm,
                aggx_hbm, aggattr_hbm, ldstl_hbm, srcl_hbm, counts_hbm,
                ebuf0, ebuf1, mldst, msrc, meid, rows0, rows1,
                arows0, arows1, accx, accattr,
                esem0, esem1, rsem0, rsem1, asem0, asem1, lsem):
    ebuf = (ebuf0, ebuf1)
    esem = (esem0, esem1)
    rowsb = (rows0, rows1)
    rsem = (rsem0, rsem1)
    arowsb = (arows0, arows1)
    asem = (asem0, asem1)

    cid = lax.axis_index("c")
    sid = lax.axis_index("s")
    wid = sid * 2 + cid
    lo = wid * NPT
    hi = lo + NPT
    regbase = wid * REG
    lane = lax.iota(jnp.int32, 16)
    neg16 = jnp.full((16,), NEG, jnp.float32)
    zero16 = jnp.zeros((16,), jnp.int32)

    def init_accx(i, c):
        accx[pl.ds(16 * i, 16)] = neg16
        return c
    lax.fori_loop(0, (NPT + 1) * D // 16, init_accx, 0)

    def init_accattr(i, c):
        accattr[pl.ds(16 * i, 16)] = neg16
        return c
    lax.fori_loop(0, NPT + 1, init_accattr, 0)

    def init_midx(i, c):
        msrc[pl.ds(16 * i, 16)] = zero16
        meid[pl.ds(16 * i, 16)] = zero16
        return c
    lax.fori_loop(0, MB // 16, init_midx, 0)

    pltpu.async_copy(edges_hbm.at[pl.ds(0, 2)], ebuf[0], esem[0])

    def spill_wait():
        pltpu.make_async_copy(mldst.at[pl.ds(0, CH)],
                              ldstl_hbm.at[pl.ds(regbase, CH)], lsem).wait()
        pltpu.make_async_copy(msrc.at[pl.ds(0, CH)],
                              srcl_hbm.at[pl.ds(regbase, CH)], lsem).wait()

    def do_chunk(c, p, tot):
        base = c * CH
        @pl.when(c + 1 < NCHUNK)
        def _():
            pltpu.async_copy(edges_hbm.at[pl.ds(2 * (c + 1), 2)],
                             ebuf[1 - p], esem[1 - p])
        pltpu.make_async_copy(edges_hbm.at[pl.ds(2 * c, 2)],
                              ebuf[p], esem[p]).wait()
        @pl.when(c > 0)
        def _():
            spill_wait()
        eb = ebuf[p]

        def scan_g(g, pos):
            v = eb[0, pl.ds(16 * g, 16)]
            m = (v >= lo) & (v < hi)
            mi = m.astype(jnp.int32)
            s = plsc.cumsum(mi)
            idx = jnp.where(m, pos + (s - mi), DUMP)
            plsc.store_scatter(mldst, [idx], v - lo)
            plsc.store_scatter(msrc, [idx], eb[1, pl.ds(16 * g, 16)])
            plsc.store_scatter(meid, [idx], lane + (base + 16 * g))
            return pos + s[15]
        nmatch = lax.fori_loop(0, CH // 16, scan_g, 0)
        mldst[pl.ds(nmatch, 16)] = jnp.full((16,), NPT, jnp.int32)
        msrc[pl.ds(nmatch, 16)] = zero16
        nmatch8 = ((nmatch + 7) // 8) * 8
        nsb = (nmatch + SB - 1) // SB

        spoff = pl.multiple_of(regbase + tot, 8)
        pltpu.async_copy(mldst.at[pl.ds(0, CH)],
                         ldstl_hbm.at[pl.ds(spoff, CH)], lsem)
        pltpu.async_copy(msrc.at[pl.ds(0, CH)],
                         srcl_hbm.at[pl.ds(spoff, CH)], lsem)

        def issue(sb, q):
            off = sb * SB
            pltpu.async_copy(feats_hbm.at[msrc.at[pl.ds(off, SB)]],
                             rowsb[q], rsem[q])
            pltpu.async_copy(attr_hbm.at[meid.at[pl.ds(off, SB)]],
                             arowsb[q], asem[q])

        def consume(sb, q):
            off = sb * SB
            cnt = jnp.minimum(nmatch - off, SB)
            pltpu.make_async_copy(feats_hbm.at[msrc.at[pl.ds(off, SB)]],
                                  rowsb[q], rsem[q]).wait()
            pltpu.make_async_copy(attr_hbm.at[meid.at[pl.ds(off, SB)]],
                                  arowsb[q], asem[q]).wait()

            def edge_body(i, carry3):
                d = mldst[pl.ds(off + i, 16)][0]
                ab = d * D
                for j in range(D // 16):
                    sl = pl.ds(ab + 16 * j, 16)
                    accx[sl] = jnp.maximum(accx[sl],
                                           rowsb[q][i, pl.ds(16 * j, 16)])
                sa = pl.ds(d * 16, 16)
                accattr[sa] = jnp.maximum(accattr[sa],
                                          arowsb[q][i, pl.ds(0, 16)])
                return carry3
            lax.fori_loop(0, cnt, edge_body, 0)

        @pl.when(nsb > 0)
        def _():
            issue(0, 0)

        def sb_pair(h, carry2):
            for q in range(2):
                sb = 2 * h + q

                @pl.when(sb < nsb)
                def _():
                    @pl.when(sb + 1 < nsb)
                    def _():
                        issue(sb + 1, 1 - q)
                    consume(sb, q)
            return carry2
        lax.fori_loop(0, (nsb + 1) // 2, sb_pair, 0)
        return tot + nmatch8

    def chunk_pair(hc, tot):
        for p in range(2):
            tot = do_chunk(2 * hc + p, p, tot)
        return tot
    total = lax.fori_loop(0, NCHUNK // 2, chunk_pair, 0)

    spill_wait()
    tailoff = pl.multiple_of(regbase + total, 8)
    pltpu.async_copy(mldst.at[pl.ds(0, CH)],
                     ldstl_hbm.at[pl.ds(tailoff, CH)], lsem)
    pltpu.async_copy(msrc.at[pl.ds(0, CH)],
                     srcl_hbm.at[pl.ds(tailoff, CH)], lsem)
    spill_wait()
    mldst[pl.ds(0, 16)] = zero16 + total
    pltpu.sync_copy(mldst.at[pl.ds(0, 16)],
                    counts_hbm.at[pl.ds(wid * 16, 16)])

    pltpu.sync_copy(accx.at[pl.ds(0, NPT * D)],
                    aggx_hbm.at[pl.ds(lo * D, NPT * D)])
    pltpu.sync_copy(accattr.at[pl.ds(0, NPT * 16)],
                    aggattr_hbm.at[pl.ds(lo * 16, NPT * 16)])


def _sc_l2_body(feats_hbm, ldstl_hbm, srcl_hbm, counts_hbm, aggx_hbm,
                lbd0, lbd1, lbs0, lbs1, rows0, rows1, accx,
                lsem0, lsem1, rsem0, rsem1):
    lbd = (lbd0, lbd1)
    lbs = (lbs0, lbs1)
    lsem = (lsem0, lsem1)
    rowsb = (rows0, rows1)
    rsem = (rsem0, rsem1)

    cid = lax.axis_index("c")
    sid = lax.axis_index("s")
    wid = sid * 2 + cid
    lo = wid * NPT
    regbase = wid * REG
    neg16 = jnp.full((16,), NEG, jnp.float32)

    def init_accx(i, c):
        accx[pl.ds(16 * i, 16)] = neg16
        return c
    lax.fori_loop(0, (NPT + 1) * D // 16, init_accx, 0)

    pltpu.sync_copy(counts_hbm.at[pl.ds(wid * 16, 16)],
                    lbd0.at[pl.ds(0, 16)])
    total = lbd0[pl.ds(0, 16)][0]
    nch = (total + CH - 1) // CH

    def load_chunk(c, p):
        off = regbase + c * CH
        pltpu.async_copy(ldstl_hbm.at[pl.ds(off, CH)], lbd[p].at[pl.ds(0, CH)],
                         lsem[p])
        pltpu.async_copy(srcl_hbm.at[pl.ds(off, CH)], lbs[p], lsem[p])

    def wait_chunk(c, p):
        off = regbase + c * CH
        pltpu.make_async_copy(ldstl_hbm.at[pl.ds(off, CH)],
                              lbd[p].at[pl.ds(0, CH)], lsem[p]).wait()
        pltpu.make_async_copy(srcl_hbm.at[pl.ds(off, CH)],
                              lbs[p], lsem[p]).wait()

    @pl.when(nch > 0)
    def _():
        load_chunk(0, 0)

    def do_chunk(c, p):
        @pl.when(c + 1 < nch)
        def _():
            load_chunk(c + 1, 1 - p)
        wait_chunk(c, p)
        nmatch = jnp.minimum(total - c * CH, CH)
        nsb = (nmatch + SB - 1) // SB

        def issue(sb, q):
            pltpu.async_copy(feats_hbm.at[lbs[p].at[pl.ds(sb * SB, SB)]],
                             rowsb[q], rsem[q])

        def consume(sb, q):
            off = sb * SB
            cnt = jnp.minimum(nmatch - off, SB)
            pltpu.make_async_copy(feats_hbm.at[lbs[p].at[pl.ds(off, SB)]],
                                  rowsb[q], rsem[q]).wait()

            def edge_body(i, carry3):
                d = lbd[p][pl.ds(off + i, 16)][0]
                ab = d * D
                for j in range(D // 16):
                    sl = pl.ds(ab + 16 * j, 16)
                    accx[sl] = jnp.maximum(accx[sl],
                                           rowsb[q][i, pl.ds(16 * j, 16)])
                return carry3
            lax.fori_loop(0, cnt, edge_body, 0)

        @pl.when(nsb > 0)
        def _():
            issue(0, 0)

        def sb_pair(h, carry2):
            for q in range(2):
                sb = 2 * h + q

                @pl.when(sb < nsb)
                def _():
                    @pl.when(sb + 1 < nsb)
                    def _():
                        issue(sb + 1, 1 - q)
                    consume(sb, q)
            return carry2
        lax.fori_loop(0, (nsb + 1) // 2, sb_pair, 0)

    def chunk_pair(hc, carry):
        for p in range(2):
            c = 2 * hc + p

            @pl.when(c < nch)
            def _():
                do_chunk(c, p)
        return carry
    lax.fori_loop(0, (nch + 1) // 2, chunk_pair, 0)

    pltpu.sync_copy(accx.at[pl.ds(0, NPT * D)],
                    aggx_hbm.at[pl.ds(lo * D, NPT * D)])


def _sc_layer1(feats, edges, attr16):
    mesh = plsc.VectorSubcoreMesh(core_axis_name="c", subcore_axis_name="s")
    params = pltpu.CompilerParams(needs_layout_passes=False,
                                  use_tc_tiling_on_sc=False)
    out_type = [jax.ShapeDtypeStruct((NPAD * D,), jnp.float32),
                jax.ShapeDtypeStruct((NPAD * 16,), jnp.float32),
                jax.ShapeDtypeStruct((TILES * REG,), jnp.int32),
                jax.ShapeDtypeStruct((TILES * REG,), jnp.int32),
                jax.ShapeDtypeStruct((TILES * 16,), jnp.int32)]
    scratch = [
        pltpu.VMEM((2, CH), jnp.int32),
        pltpu.VMEM((2, CH), jnp.int32),
        pltpu.VMEM((MB,), jnp.int32),
        pltpu.VMEM((MB,), jnp.int32),
        pltpu.VMEM((MB,), jnp.int32),
        pltpu.VMEM((SB, D), jnp.float32),
        pltpu.VMEM((SB, D), jnp.float32),
        pltpu.VMEM((SB, 16), jnp.float32),
        pltpu.VMEM((SB, 16), jnp.float32),
        pltpu.VMEM(((NPT + 1) * D,), jnp.float32),
        pltpu.VMEM(((NPT + 1) * 16,), jnp.float32),
        pltpu.SemaphoreType.DMA,
        pltpu.SemaphoreType.DMA,
        pltpu.SemaphoreType.DMA,
        pltpu.SemaphoreType.DMA,
        pltpu.SemaphoreType.DMA,
        pltpu.SemaphoreType.DMA,
        pltpu.SemaphoreType.DMA,
    ]
    f = pl.kernel(_sc_l1_body, out_type=out_type, mesh=mesh,
                  scratch_types=scratch, compiler_params=params)
    return f(edges, feats, attr16)


def _sc_layer2(feats, ldstl, srcl, counts):
    mesh = plsc.VectorSubcoreMesh(core_axis_name="c", subcore_axis_name="s")
    params = pltpu.CompilerParams(needs_layout_passes=False,
                                  use_tc_tiling_on_sc=False)
    out_type = jax.ShapeDtypeStruct((NPAD * D,), jnp.float32)
    scratch = [
        pltpu.VMEM((CH + 16,), jnp.int32),
        pltpu.VMEM((CH + 16,), jnp.int32),
        pltpu.VMEM((CH,), jnp.int32),
        pltpu.VMEM((CH,), jnp.int32),
        pltpu.VMEM((SB, D), jnp.float32),
        pltpu.VMEM((SB, D), jnp.float32),
        pltpu.VMEM(((NPT + 1) * D,), jnp.float32),
        pltpu.SemaphoreType.DMA,
        pltpu.SemaphoreType.DMA,
        pltpu.SemaphoreType.DMA,
        pltpu.SemaphoreType.DMA,
    ]
    f = pl.kernel(_sc_l2_body, out_type=out_type, mesh=mesh,
                  scratch_types=scratch, compiler_params=params)
    return f(feats, ldstl, srcl, counts)


def _fix(a):
    return jnp.where(jnp.isfinite(a) & (a > NEG), a, 0.0)


def _dense1_body(ax_ref, aa_ref, wx_ref, wa_ref, b_ref, g_ref, bt_ref, o_ref):
    ax = _fix(ax_ref[...])
    aa = _fix(aa_ref[...])
    h = (jnp.dot(ax, wx_ref[...], preferred_element_type=jnp.float32)
         + jnp.dot(aa, wa_ref[...], preferred_element_type=jnp.float32)
         + b_ref[...])
    row = lax.broadcasted_iota(jnp.int32, (NPAD, 1), 0)
    mask = (row < N).astype(jnp.float32)
    mean = jnp.sum(h * mask, axis=0, keepdims=True) * (1.0 / N)
    d = (h - mean) * mask
    var = jnp.sum(d * d, axis=0, keepdims=True) * (1.0 / N)
    hn = (h - mean) * lax.rsqrt(var + EPS) * g_ref[...] + bt_ref[...]
    o_ref[...] = jnp.maximum(hn, 0.0) * mask


def _dense2_body(ax_ref, aa_ref, wx_ref, wa_ref, b_ref, g_ref, bt_ref,
                 wout_ref, bout_ref, batch_ref, o_ref):
    ax = _fix(ax_ref[...])
    aa = _fix(aa_ref[...])
    h = (jnp.dot(ax, wx_ref[...], preferred_element_type=jnp.float32)
         + jnp.dot(aa, wa_ref[...], preferred_element_type=jnp.float32)
         + b_ref[...])
    row = lax.broadcasted_iota(jnp.int32, (NPAD, 1), 0)
    mask = (row < N).astype(jnp.float32)
    mean = jnp.sum(h * mask, axis=0, keepdims=True) * (1.0 / N)
    d = (h - mean) * mask
    var = jnp.sum(d * d, axis=0, keepdims=True) * (1.0 / N)
    hn = (h - mean) * lax.rsqrt(var + EPS) * g_ref[...] + bt_ref[...]
    h1 = jnp.maximum(hn, 0.0)
    valid = row < N
    b = batch_ref[...]
    cols = []
    for gidx in range(G):
        sel = jnp.where((b == gidx) & valid, h1, -jnp.inf)
        cols.append(jnp.max(sel, axis=0, keepdims=True))
    pooled = jnp.concatenate(cols, axis=0)
    pooled = jnp.where(jnp.isfinite(pooled), pooled, 0.0)
    o_ref[...] = (jnp.dot(pooled, wout_ref[...],
                          preferred_element_type=jnp.float32) + bout_ref[...])


def kernel(x, edge_index, edge_attr, batch, W0, b0, g0, bt0, W1, b1, g1, bt1,
           Wout, bout):
    dst = edge_index[1].reshape(NCHUNK, 1, CH)
    src = edge_index[0].reshape(NCHUNK, 1, CH)
    edges = jnp.concatenate([dst, src], axis=1).reshape(2 * NCHUNK, CH)
    attr16 = jnp.pad(edge_attr, ((0, 0), (0, 10)))
    xpad = jnp.pad(x, ((0, NPAD - N), (0, 0)))
    batch2d = jnp.pad(batch, (0, NPAD - N), constant_values=G).reshape(NPAD, 1)

    aggx0_f, aggattr_f, ldstl, srcl, counts = _sc_layer1(xpad, edges, attr16)
    aggx0 = aggx0_f.reshape(NPAD, D)
    aggattr = aggattr_f.reshape(NPAD, 16)

    W0x = W0[:D]
    W0a = jnp.pad(W0[D:], ((0, 10), (0, 0)))
    h0 = pl.pallas_call(
        _dense1_body,
        out_shape=jax.ShapeDtypeStruct((NPAD, H), jnp.float32),
    )(aggx0, aggattr, W0x, W0a, b0.reshape(1, H), g0.reshape(1, H),
      bt0.reshape(1, H))

    aggx1 = _sc_layer2(h0, ldstl, srcl, counts).reshape(NPAD, D)

    W1x = W1[:H]
    W1a = jnp.pad(W1[H:], ((0, 10), (0, 0)))
    out = pl.pallas_call(
        _dense2_body,
        out_shape=jax.ShapeDtypeStruct((G, 1), jnp.float32),
    )(aggx1, aggattr, W1x, W1a, b1.reshape(1, H), g1.reshape(1, H),
      bt1.reshape(1, H), Wout, bout.reshape(1, 1), batch2d)
    return out


# speedup vs baseline: 2.0353x; 1.0321x over previous
import functools

import jax
import jax.numpy as jnp
from jax import lax
from jax.experimental import pallas as pl
from jax.experimental.pallas import tpu as pltpu
from jax.experimental.pallas import tpu_sc as plsc

N = 10000
E = 320000
D = 128
H = 128
G = 64
EPS = 1e-5

TILES = 32
NPT = 313
NPAD = TILES * NPT
CH = 6400
NCHUNK = E // CH
SB = 128
MB = CH + SB
DUMP = MB - 16
REG = E + 2 * CH
NEG = -3.0e38


def _sc_l1_body(edges_hbm, feats_hbm, attr_hbm,
                aggx_hbm, aggattr_hbm, ldstl_hbm, srcl_hbm, counts_hbm,
                ebuf0, ebuf1, mldst, msrc, meid, rows0, rows1,
                arows0, arows1, accx, accattr,
                esem0, esem1, rsem0, rsem1, asem0, asem1, lsem):
    ebuf = (ebuf0, ebuf1)
    esem = (esem0, esem1)
    rowsb = (rows0, rows1)
    rsem = (rsem0, rsem1)
    arowsb = (arows0, arows1)
    asem = (asem0, asem1)

    cid = lax.axis_index("c")
    sid = lax.axis_index("s")
    wid = sid * 2 + cid
    lo = wid * NPT
    hi = lo + NPT
    regbase = wid * REG
    lane = lax.iota(jnp.int32, 16)
    neg16 = jnp.full((16,), NEG, jnp.float32)
    zero16 = jnp.zeros((16,), jnp.int32)

    def init_accx(i, c):
        accx[pl.ds(16 * i, 16)] = neg16
        return c
    lax.fori_loop(0, (NPT + 1) * D // 16, init_accx, 0)

    def init_accattr(i, c):
        accattr[pl.ds(16 * i, 16)] = neg16
        return c
    lax.fori_loop(0, NPT + 1, init_accattr, 0)

    def init_midx(i, c):
        msrc[pl.ds(16 * i, 16)] = zero16
        meid[pl.ds(16 * i, 16)] = zero16
        return c
    lax.fori_loop(0, MB // 16, init_midx, 0)

    pltpu.async_copy(edges_hbm.at[pl.ds(0, 2)], ebuf[0], esem[0])

    def spill_wait():
        pltpu.make_async_copy(mldst.at[pl.ds(0, CH)],
                              ldstl_hbm.at[pl.ds(regbase, CH)], lsem).wait()
        pltpu.make_async_copy(msrc.at[pl.ds(0, CH)],
                              srcl_hbm.at[pl.ds(regbase, CH)], lsem).wait()

    def do_chunk(c, p, tot):
        base = c * CH
        @pl.when(c + 1 < NCHUNK)
        def _():
            pltpu.async_copy(edges_hbm.at[pl.ds(2 * (c + 1), 2)],
                             ebuf[1 - p], esem[1 - p])
        pltpu.make_async_copy(edges_hbm.at[pl.ds(2 * c, 2)],
                              ebuf[p], esem[p]).wait()
        @pl.when(c > 0)
        def _():
            spill_wait()
        eb = ebuf[p]

        def scan_g(g, pos):
            v = eb[0, pl.ds(16 * g, 16)]
            m = (v >= lo) & (v < hi)
            mi = m.astype(jnp.int32)
            s = plsc.cumsum(mi)
            idx = jnp.where(m, pos + (s - mi), DUMP)
            plsc.store_scatter(mldst, [idx], v - lo)
            plsc.store_scatter(msrc, [idx], eb[1, pl.ds(16 * g, 16)])
            plsc.store_scatter(meid, [idx], lane + (base + 16 * g))
            return pos + s[15]
        nmatch = lax.fori_loop(0, CH // 16, scan_g, 0)
        mldst[pl.ds(nmatch, 16)] = jnp.full((16,), NPT, jnp.int32)
        msrc[pl.ds(nmatch, 16)] = zero16
        nmatch8 = ((nmatch + 7) // 8) * 8
        nsb = (nmatch + SB - 1) // SB

        spoff = pl.multiple_of(regbase + tot, 8)
        pltpu.async_copy(mldst.at[pl.ds(0, CH)],
                         ldstl_hbm.at[pl.ds(spoff, CH)], lsem)
        pltpu.async_copy(msrc.at[pl.ds(0, CH)],
                         srcl_hbm.at[pl.ds(spoff, CH)], lsem)

        def issue(sb, q):
            off = sb * SB
            pltpu.async_copy(feats_hbm.at[msrc.at[pl.ds(off, SB)]],
                             rowsb[q], rsem[q])
            pltpu.async_copy(attr_hbm.at[meid.at[pl.ds(off, SB)]],
                             arowsb[q], asem[q])

        def consume(sb, q):
            off = sb * SB
            cnt = jnp.minimum(nmatch - off, SB)
            pltpu.make_async_copy(feats_hbm.at[msrc.at[pl.ds(off, SB)]],
                                  rowsb[q], rsem[q]).wait()
            pltpu.make_async_copy(attr_hbm.at[meid.at[pl.ds(off, SB)]],
                                  arowsb[q], asem[q]).wait()

            def grp_body(gi, carry3):
                grp = mldst[pl.ds(off + gi * 16, 16)]
                for l in range(16):
                    d = grp[l]
                    ab = d * D
                    i = gi * 16 + l
                    for j in range(D // 16):
                        sl = pl.ds(ab + 16 * j, 16)
                        accx[sl] = jnp.maximum(
                            accx[sl], rowsb[q][i, pl.ds(16 * j, 16)])
                    sa = pl.ds(d * 16, 16)
                    accattr[sa] = jnp.maximum(accattr[sa],
                                              arowsb[q][i, pl.ds(0, 16)])
                return carry3
            lax.fori_loop(0, (cnt + 15) // 16, grp_body, 0)

        @pl.when(nsb > 0)
        def _():
            issue(0, 0)

        def sb_pair(h, carry2):
            for q in range(2):
                sb = 2 * h + q

                @pl.when(sb < nsb)
                def _():
                    @pl.when(sb + 1 < nsb)
                    def _():
                        issue(sb + 1, 1 - q)
                    consume(sb, q)
            return carry2
        lax.fori_loop(0, (nsb + 1) // 2, sb_pair, 0)
        return tot + nmatch8

    def chunk_pair(hc, tot):
        for p in range(2):
            tot = do_chunk(2 * hc + p, p, tot)
        return tot
    total = lax.fori_loop(0, NCHUNK // 2, chunk_pair, 0)

    spill_wait()
    mldst[pl.ds(0, 16)] = jnp.full((16,), NPT, jnp.int32)
    tailoff = pl.multiple_of(regbase + total, 8)
    pltpu.async_copy(mldst.at[pl.ds(0, CH)],
                     ldstl_hbm.at[pl.ds(tailoff, CH)], lsem)
    pltpu.async_copy(msrc.at[pl.ds(0, CH)],
                     srcl_hbm.at[pl.ds(tailoff, CH)], lsem)
    spill_wait()
    mldst[pl.ds(0, 16)] = zero16 + total
    pltpu.sync_copy(mldst.at[pl.ds(0, 16)],
                    counts_hbm.at[pl.ds(wid * 16, 16)])

    pltpu.sync_copy(accx.at[pl.ds(0, NPT * D)],
                    aggx_hbm.at[pl.ds(lo * D, NPT * D)])
    pltpu.sync_copy(accattr.at[pl.ds(0, NPT * 16)],
                    aggattr_hbm.at[pl.ds(lo * 16, NPT * 16)])


def _sc_l2_body(feats_hbm, ldstl_hbm, srcl_hbm, counts_hbm, aggx_hbm,
                lbd0, lbd1, lbs0, lbs1, rows0, rows1, accx,
                lsem0, lsem1, rsem0, rsem1):
    lbd = (lbd0, lbd1)
    lbs = (lbs0, lbs1)
    lsem = (lsem0, lsem1)
    rowsb = (rows0, rows1)
    rsem = (rsem0, rsem1)

    cid = lax.axis_index("c")
    sid = lax.axis_index("s")
    wid = sid * 2 + cid
    lo = wid * NPT
    regbase = wid * REG
    neg16 = jnp.full((16,), NEG, jnp.float32)

    def init_accx(i, c):
        accx[pl.ds(16 * i, 16)] = neg16
        return c
    lax.fori_loop(0, (NPT + 1) * D // 16, init_accx, 0)

    pltpu.sync_copy(counts_hbm.at[pl.ds(wid * 16, 16)],
                    lbd0.at[pl.ds(0, 16)])
    total = lbd0[pl.ds(0, 16)][0]
    nch = (total + CH - 1) // CH

    def load_chunk(c, p):
        off = regbase + c * CH
        pltpu.async_copy(ldstl_hbm.at[pl.ds(off, CH)], lbd[p].at[pl.ds(0, CH)],
                         lsem[p])
        pltpu.async_copy(srcl_hbm.at[pl.ds(off, CH)], lbs[p], lsem[p])

    def wait_chunk(c, p):
        off = regbase + c * CH
        pltpu.make_async_copy(ldstl_hbm.at[pl.ds(off, CH)],
                              lbd[p].at[pl.ds(0, CH)], lsem[p]).wait()
        pltpu.make_async_copy(srcl_hbm.at[pl.ds(off, CH)],
                              lbs[p], lsem[p]).wait()

    @pl.when(nch > 0)
    def _():
        load_chunk(0, 0)

    def do_chunk(c, p):
        @pl.when(c + 1 < nch)
        def _():
            load_chunk(c + 1, 1 - p)
        wait_chunk(c, p)
        nmatch = jnp.minimum(total - c * CH, CH)
        nsb = (nmatch + SB - 1) // SB

        def issue(sb, q):
            pltpu.async_copy(feats_hbm.at[lbs[p].at[pl.ds(sb * SB, SB)]],
                             rowsb[q], rsem[q])

        def consume(sb, q):
            off = sb * SB
            cnt = jnp.minimum(nmatch - off, SB)
            pltpu.make_async_copy(feats_hbm.at[lbs[p].at[pl.ds(off, SB)]],
                                  rowsb[q], rsem[q]).wait()

            def grp_body(gi, carry3):
                grp = lbd[p][pl.ds(off + gi * 16, 16)]
                for l in range(16):
                    d = grp[l]
                    ab = d * D
                    i = gi * 16 + l
                    for j in range(D // 16):
                        sl = pl.ds(ab + 16 * j, 16)
                        accx[sl] = jnp.maximum(
                            accx[sl], rowsb[q][i, pl.ds(16 * j, 16)])
                return carry3
            lax.fori_loop(0, (cnt + 15) // 16, grp_body, 0)

        @pl.when(nsb > 0)
        def _():
            issue(0, 0)

        def sb_pair(h, carry2):
            for q in range(2):
                sb = 2 * h + q

                @pl.when(sb < nsb)
                def _():
                    @pl.when(sb + 1 < nsb)
                    def _():
                        issue(sb + 1, 1 - q)
                    consume(sb, q)
            return carry2
        lax.fori_loop(0, (nsb + 1) // 2, sb_pair, 0)

    def chunk_pair(hc, carry):
        for p in range(2):
            c = 2 * hc + p

            @pl.when(c < nch)
            def _():
                do_chunk(c, p)
        return carry
    lax.fori_loop(0, (nch + 1) // 2, chunk_pair, 0)

    pltpu.sync_copy(accx.at[pl.ds(0, NPT * D)],
                    aggx_hbm.at[pl.ds(lo * D, NPT * D)])


def _sc_layer1(feats, edges, attr16):
    mesh = plsc.VectorSubcoreMesh(core_axis_name="c", subcore_axis_name="s")
    params = pltpu.CompilerParams(needs_layout_passes=False,
                                  use_tc_tiling_on_sc=False)
    out_type = [jax.ShapeDtypeStruct((NPAD * D,), jnp.float32),
                jax.ShapeDtypeStruct((NPAD * 16,), jnp.float32),
                jax.ShapeDtypeStruct((TILES * REG,), jnp.int32),
                jax.ShapeDtypeStruct((TILES * REG,), jnp.int32),
                jax.ShapeDtypeStruct((TILES * 16,), jnp.int32)]
    scratch = [
        pltpu.VMEM((2, CH), jnp.int32),
        pltpu.VMEM((2, CH), jnp.int32),
        pltpu.VMEM((MB,), jnp.int32),
        pltpu.VMEM((MB,), jnp.int32),
        pltpu.VMEM((MB,), jnp.int32),
        pltpu.VMEM((SB, D), jnp.float32),
        pltpu.VMEM((SB, D), jnp.float32),
        pltpu.VMEM((SB, 16), jnp.float32),
        pltpu.VMEM((SB, 16), jnp.float32),
        pltpu.VMEM(((NPT + 1) * D,), jnp.float32),
        pltpu.VMEM(((NPT + 1) * 16,), jnp.float32),
        pltpu.SemaphoreType.DMA,
        pltpu.SemaphoreType.DMA,
        pltpu.SemaphoreType.DMA,
        pltpu.SemaphoreType.DMA,
        pltpu.SemaphoreType.DMA,
        pltpu.SemaphoreType.DMA,
        pltpu.SemaphoreType.DMA,
    ]
    f = pl.kernel(_sc_l1_body, out_type=out_type, mesh=mesh,
                  scratch_types=scratch, compiler_params=params)
    return f(edges, feats, attr16)


def _sc_layer2(feats, ldstl, srcl, counts):
    mesh = plsc.VectorSubcoreMesh(core_axis_name="c", subcore_axis_name="s")
    params = pltpu.CompilerParams(needs_layout_passes=False,
                                  use_tc_tiling_on_sc=False)
    out_type = jax.ShapeDtypeStruct((NPAD * D,), jnp.float32)
    scratch = [
        pltpu.VMEM((CH + 16,), jnp.int32),
        pltpu.VMEM((CH + 16,), jnp.int32),
        pltpu.VMEM((CH,), jnp.int32),
        pltpu.VMEM((CH,), jnp.int32),
        pltpu.VMEM((SB, D), jnp.float32),
        pltpu.VMEM((SB, D), jnp.float32),
        pltpu.VMEM(((NPT + 1) * D,), jnp.float32),
        pltpu.SemaphoreType.DMA,
        pltpu.SemaphoreType.DMA,
        pltpu.SemaphoreType.DMA,
        pltpu.SemaphoreType.DMA,
    ]
    f = pl.kernel(_sc_l2_body, out_type=out_type, mesh=mesh,
                  scratch_types=scratch, compiler_params=params)
    return f(feats, ldstl, srcl, counts)


def _fix(a):
    return jnp.where(jnp.isfinite(a) & (a > NEG), a, 0.0)


def _dense1_body(ax_ref, aa_ref, wx_ref, wa_ref, b_ref, g_ref, bt_ref, o_ref):
    ax = _fix(ax_ref[...])
    aa = _fix(aa_ref[...])
    h = (jnp.dot(ax, wx_ref[...], preferred_element_type=jnp.float32)
         + jnp.dot(aa, wa_ref[...], preferred_element_type=jnp.float32)
         + b_ref[...])
    row = lax.broadcasted_iota(jnp.int32, (NPAD, 1), 0)
    mask = (row < N).astype(jnp.float32)
    mean = jnp.sum(h * mask, axis=0, keepdims=True) * (1.0 / N)
    d = (h - mean) * mask
    var = jnp.sum(d * d, axis=0, keepdims=True) * (1.0 / N)
    hn = (h - mean) * lax.rsqrt(var + EPS) * g_ref[...] + bt_ref[...]
    o_ref[...] = jnp.maximum(hn, 0.0) * mask


def _dense2_body(ax_ref, aa_ref, wx_ref, wa_ref, b_ref, g_ref, bt_ref,
                 wout_ref, bout_ref, batch_ref, o_ref):
    ax = _fix(ax_ref[...])
    aa = _fix(aa_ref[...])
    h = (jnp.dot(ax, wx_ref[...], preferred_element_type=jnp.float32)
         + jnp.dot(aa, wa_ref[...], preferred_element_type=jnp.float32)
         + b_ref[...])
    row = lax.broadcasted_iota(jnp.int32, (NPAD, 1), 0)
    mask = (row < N).astype(jnp.float32)
    mean = jnp.sum(h * mask, axis=0, keepdims=True) * (1.0 / N)
    d = (h - mean) * mask
    var = jnp.sum(d * d, axis=0, keepdims=True) * (1.0 / N)
    hn = (h - mean) * lax.rsqrt(var + EPS) * g_ref[...] + bt_ref[...]
    h1 = jnp.maximum(hn, 0.0)
    valid = row < N
    b = batch_ref[...]
    cols = []
    for gidx in range(G):
        sel = jnp.where((b == gidx) & valid, h1, -jnp.inf)
        cols.append(jnp.max(sel, axis=0, keepdims=True))
    pooled = jnp.concatenate(cols, axis=0)
    pooled = jnp.where(jnp.isfinite(pooled), pooled, 0.0)
    o_ref[...] = (jnp.dot(pooled, wout_ref[...],
                          preferred_element_type=jnp.float32) + bout_ref[...])


def kernel(x, edge_index, edge_attr, batch, W0, b0, g0, bt0, W1, b1, g1, bt1,
           Wout, bout):
    dst = edge_index[1].reshape(NCHUNK, 1, CH)
    src = edge_index[0].reshape(NCHUNK, 1, CH)
    edges = jnp.concatenate([dst, src], axis=1).reshape(2 * NCHUNK, CH)
    attr16 = jnp.pad(edge_attr, ((0, 0), (0, 10)))
    xpad = jnp.pad(x, ((0, NPAD - N), (0, 0)))
    batch2d = jnp.pad(batch, (0, NPAD - N), constant_values=G).reshape(NPAD, 1)

    aggx0_f, aggattr_f, ldstl, srcl, counts = _sc_layer1(xpad, edges, attr16)
    aggx0 = aggx0_f.reshape(NPAD, D)
    aggattr = aggattr_f.reshape(NPAD, 16)

    W0x = W0[:D]
    W0a = jnp.pad(W0[D:], ((0, 10), (0, 0)))
    h0 = pl.pallas_call(
        _dense1_body,
        out_shape=jax.ShapeDtypeStruct((NPAD, H), jnp.float32),
    )(aggx0, aggattr, W0x, W0a, b0.reshape(1, H), g0.reshape(1, H),
      bt0.reshape(1, H))

    aggx1 = _sc_layer2(h0, ldstl, srcl, counts).reshape(NPAD, D)

    W1x = W1[:H]
    W1a = jnp.pad(W1[H:], ((0, 10), (0, 0)))
    out = pl.pallas_call(
        _dense2_body,
        out_shape=jax.ShapeDtypeStruct((G, 1), jnp.float32),
    )(aggx1, aggattr, W1x, W1a, b1.reshape(1, H), g1.reshape(1, H),
      bt1.reshape(1, H), Wout, bout.reshape(1, 1), batch2d)
    return out


# speedup vs baseline: 2.6711x; 1.3124x over previous
import functools

import jax
import jax.numpy as jnp
from jax import lax
from jax.experimental import pallas as pl
from jax.experimental.pallas import tpu as pltpu
from jax.experimental.pallas import tpu_sc as plsc

N = 10000
E = 320000
D = 128
H = 128
G = 64
EPS = 1e-5

TILES = 32
NPT = 313
NPAD = TILES * NPT
CH = 6400
NCHUNK = E // CH
SB = 64
MB = CH + SB
DUMP = MB - 16
REG = E + 2 * CH
NEG = -3.0e38


def _sc_l1_body(edges_hbm, feats_hbm, attr_hbm,
                aggx_hbm, aggattr_hbm, ldstl_hbm, srcl_hbm, counts_hbm,
                ebuf0, ebuf1, mldst, msrc, meid,
                rows0, rows1, rows2, rows3,
                arows0, arows1, arows2, arows3, accx, accattr,
                esem0, esem1, rsem0, rsem1, rsem2, rsem3,
                asem0, asem1, asem2, asem3, lsem):
    ebuf = (ebuf0, ebuf1)
    esem = (esem0, esem1)
    rowsb = (rows0, rows1, rows2, rows3)
    rsem = (rsem0, rsem1, rsem2, rsem3)
    arowsb = (arows0, arows1, arows2, arows3)
    asem = (asem0, asem1, asem2, asem3)

    cid = lax.axis_index("c")
    sid = lax.axis_index("s")
    wid = sid * 2 + cid
    lo = wid * NPT
    hi = lo + NPT
    regbase = wid * REG
    lane = lax.iota(jnp.int32, 16)
    neg16 = jnp.full((16,), NEG, jnp.float32)
    zero16 = jnp.zeros((16,), jnp.int32)

    def init_accx(i, c):
        accx[pl.ds(16 * i, 16)] = neg16
        return c
    lax.fori_loop(0, (NPT + 1) * D // 16, init_accx, 0)

    def init_accattr(i, c):
        accattr[pl.ds(16 * i, 16)] = neg16
        return c
    lax.fori_loop(0, NPT + 1, init_accattr, 0)

    def init_midx(i, c):
        msrc[pl.ds(16 * i, 16)] = zero16
        meid[pl.ds(16 * i, 16)] = zero16
        return c
    lax.fori_loop(0, MB // 16, init_midx, 0)

    pltpu.async_copy(edges_hbm.at[pl.ds(0, 2)], ebuf[0], esem[0])

    def spill_wait():
        pltpu.make_async_copy(mldst.at[pl.ds(0, CH)],
                              ldstl_hbm.at[pl.ds(regbase, CH)], lsem).wait()
        pltpu.make_async_copy(msrc.at[pl.ds(0, CH)],
                              srcl_hbm.at[pl.ds(regbase, CH)], lsem).wait()

    def do_chunk(c, p, tot):
        base = c * CH
        @pl.when(c + 1 < NCHUNK)
        def _():
            pltpu.async_copy(edges_hbm.at[pl.ds(2 * (c + 1), 2)],
                             ebuf[1 - p], esem[1 - p])
        pltpu.make_async_copy(edges_hbm.at[pl.ds(2 * c, 2)],
                              ebuf[p], esem[p]).wait()
        @pl.when(c > 0)
        def _():
            spill_wait()
        eb = ebuf[p]

        def scan_g(g, pos):
            v = eb[0, pl.ds(16 * g, 16)]
            m = (v >= lo) & (v < hi)
            mi = m.astype(jnp.int32)
            s = plsc.cumsum(mi)
            idx = jnp.where(m, pos + (s - mi), DUMP)
            plsc.store_scatter(mldst, [idx], v - lo)
            plsc.store_scatter(msrc, [idx], eb[1, pl.ds(16 * g, 16)])
            plsc.store_scatter(meid, [idx], lane + (base + 16 * g))
            return pos + s[15]
        nmatch = lax.fori_loop(0, CH // 16, scan_g, 0)
        mldst[pl.ds(nmatch, 16)] = jnp.full((16,), NPT, jnp.int32)
        msrc[pl.ds(nmatch, 16)] = zero16
        nmatch8 = ((nmatch + 7) // 8) * 8
        nsb = (nmatch + SB - 1) // SB

        spoff = pl.multiple_of(regbase + tot, 8)
        pltpu.async_copy(mldst.at[pl.ds(0, CH)],
                         ldstl_hbm.at[pl.ds(spoff, CH)], lsem)
        pltpu.async_copy(msrc.at[pl.ds(0, CH)],
                         srcl_hbm.at[pl.ds(spoff, CH)], lsem)

        def issue(sb, q):
            off = sb * SB
            pltpu.async_copy(feats_hbm.at[msrc.at[pl.ds(off, SB)]],
                             rowsb[q], rsem[q])
            pltpu.async_copy(attr_hbm.at[meid.at[pl.ds(off, SB)]],
                             arowsb[q], asem[q])

        def consume(sb, q):
            off = sb * SB
            cnt = jnp.minimum(nmatch - off, SB)
            pltpu.make_async_copy(feats_hbm.at[msrc.at[pl.ds(off, SB)]],
                                  rowsb[q], rsem[q]).wait()
            pltpu.make_async_copy(attr_hbm.at[meid.at[pl.ds(off, SB)]],
                                  arowsb[q], asem[q]).wait()

            def grp_body(gi, carry3):
                grp = mldst[pl.ds(off + gi * 16, 16)]
                for l in range(16):
                    d = grp[l]
                    ab = d * D
                    i = gi * 16 + l
                    for j in range(D // 16):
                        sl = pl.ds(ab + 16 * j, 16)
                        accx[sl] = jnp.maximum(
                            accx[sl], rowsb[q][i, pl.ds(16 * j, 16)])
                    sa = pl.ds(d * 16, 16)
                    accattr[sa] = jnp.maximum(accattr[sa],
                                              arowsb[q][i, pl.ds(0, 16)])
                return carry3
            lax.fori_loop(0, (cnt + 15) // 16, grp_body, 0)

        for w in range(3):
            @pl.when(w < nsb)
            def _(w=w):
                issue(w, w)

        def sb_quad(h, carry2):
            for q in range(4):
                sb = 4 * h + q

                @pl.when(sb < nsb)
                def _(sb=sb, q=q):
                    @pl.when(sb + 3 < nsb)
                    def _():
                        issue(sb + 3, (q + 3) % 4)
                    consume(sb, q)
            return carry2
        lax.fori_loop(0, (nsb + 3) // 4, sb_quad, 0)
        return tot + nmatch8

    def chunk_pair(hc, tot):
        for p in range(2):
            tot = do_chunk(2 * hc + p, p, tot)
        return tot
    total = lax.fori_loop(0, NCHUNK // 2, chunk_pair, 0)

    spill_wait()
    mldst[pl.ds(0, 16)] = jnp.full((16,), NPT, jnp.int32)
    tailoff = pl.multiple_of(regbase + total, 8)
    pltpu.async_copy(mldst.at[pl.ds(0, CH)],
                     ldstl_hbm.at[pl.ds(tailoff, CH)], lsem)
    pltpu.async_copy(msrc.at[pl.ds(0, CH)],
                     srcl_hbm.at[pl.ds(tailoff, CH)], lsem)
    spill_wait()
    mldst[pl.ds(0, 16)] = zero16 + total
    pltpu.sync_copy(mldst.at[pl.ds(0, 16)],
                    counts_hbm.at[pl.ds(wid * 16, 16)])

    pltpu.sync_copy(accx.at[pl.ds(0, NPT * D)],
                    aggx_hbm.at[pl.ds(lo * D, NPT * D)])
    pltpu.sync_copy(accattr.at[pl.ds(0, NPT * 16)],
                    aggattr_hbm.at[pl.ds(lo * 16, NPT * 16)])


def _sc_l2_body(feats_hbm, ldstl_hbm, srcl_hbm, counts_hbm, aggx_hbm,
                lbd0, lbd1, lbs0, lbs1,
                rows0, rows1, rows2, rows3, accx,
                lsem0, lsem1, rsem0, rsem1, rsem2, rsem3):
    lbd = (lbd0, lbd1)
    lbs = (lbs0, lbs1)
    lsem = (lsem0, lsem1)
    rowsb = (rows0, rows1, rows2, rows3)
    rsem = (rsem0, rsem1, rsem2, rsem3)

    cid = lax.axis_index("c")
    sid = lax.axis_index("s")
    wid = sid * 2 + cid
    lo = wid * NPT
    regbase = wid * REG
    neg16 = jnp.full((16,), NEG, jnp.float32)

    def init_accx(i, c):
        accx[pl.ds(16 * i, 16)] = neg16
        return c
    lax.fori_loop(0, (NPT + 1) * D // 16, init_accx, 0)

    pltpu.sync_copy(counts_hbm.at[pl.ds(wid * 16, 16)],
                    lbd0.at[pl.ds(0, 16)])
    total = lbd0[pl.ds(0, 16)][0]
    nch = (total + CH - 1) // CH

    def load_chunk(c, p):
        off = regbase + c * CH
        pltpu.async_copy(ldstl_hbm.at[pl.ds(off, CH)], lbd[p].at[pl.ds(0, CH)],
                         lsem[p])
        pltpu.async_copy(srcl_hbm.at[pl.ds(off, CH)], lbs[p], lsem[p])

    def wait_chunk(c, p):
        off = regbase + c * CH
        pltpu.make_async_copy(ldstl_hbm.at[pl.ds(off, CH)],
                              lbd[p].at[pl.ds(0, CH)], lsem[p]).wait()
        pltpu.make_async_copy(srcl_hbm.at[pl.ds(off, CH)],
                              lbs[p], lsem[p]).wait()

    @pl.when(nch > 0)
    def _():
        load_chunk(0, 0)

    def do_chunk(c, p):
        @pl.when(c + 1 < nch)
        def _():
            load_chunk(c + 1, 1 - p)
        wait_chunk(c, p)
        nmatch = jnp.minimum(total - c * CH, CH)
        nsb = (nmatch + SB - 1) // SB

        def issue(sb, q):
            pltpu.async_copy(feats_hbm.at[lbs[p].at[pl.ds(sb * SB, SB)]],
                             rowsb[q], rsem[q])

        def consume(sb, q):
            off = sb * SB
            cnt = jnp.minimum(nmatch - off, SB)
            pltpu.make_async_copy(feats_hbm.at[lbs[p].at[pl.ds(off, SB)]],
                                  rowsb[q], rsem[q]).wait()

            def grp_body(gi, carry3):
                grp = lbd[p][pl.ds(off + gi * 16, 16)]
                for l in range(16):
                    d = grp[l]
                    ab = d * D
                    i = gi * 16 + l
                    for j in range(D // 16):
                        sl = pl.ds(ab + 16 * j, 16)
                        accx[sl] = jnp.maximum(
                            accx[sl], rowsb[q][i, pl.ds(16 * j, 16)])
                return carry3
            lax.fori_loop(0, (cnt + 15) // 16, grp_body, 0)

        for w in range(3):
            @pl.when(w < nsb)
            def _(w=w):
                issue(w, w)

        def sb_quad(h, carry2):
            for q in range(4):
                sb = 4 * h + q

                @pl.when(sb < nsb)
                def _(sb=sb, q=q):
                    @pl.when(sb + 3 < nsb)
                    def _():
                        issue(sb + 3, (q + 3) % 4)
                    consume(sb, q)
            return carry2
        lax.fori_loop(0, (nsb + 3) // 4, sb_quad, 0)

    def chunk_pair(hc, carry):
        for p in range(2):
            c = 2 * hc + p

            @pl.when(c < nch)
            def _():
                do_chunk(c, p)
        return carry
    lax.fori_loop(0, (nch + 1) // 2, chunk_pair, 0)

    pltpu.sync_copy(accx.at[pl.ds(0, NPT * D)],
                    aggx_hbm.at[pl.ds(lo * D, NPT * D)])


def _sc_layer1(feats, edges, attr16):
    mesh = plsc.VectorSubcoreMesh(core_axis_name="c", subcore_axis_name="s")
    params = pltpu.CompilerParams(needs_layout_passes=False,
                                  use_tc_tiling_on_sc=False)
    out_type = [jax.ShapeDtypeStruct((NPAD * D,), jnp.float32),
                jax.ShapeDtypeStruct((NPAD * 16,), jnp.float32),
                jax.ShapeDtypeStruct((TILES * REG,), jnp.int32),
                jax.ShapeDtypeStruct((TILES * REG,), jnp.int32),
                jax.ShapeDtypeStruct((TILES * 16,), jnp.int32)]
    scratch = [
        pltpu.VMEM((2, CH), jnp.int32),
        pltpu.VMEM((2, CH), jnp.int32),
        pltpu.VMEM((MB,), jnp.int32),
        pltpu.VMEM((MB,), jnp.int32),
        pltpu.VMEM((MB,), jnp.int32),
        pltpu.VMEM((SB, D), jnp.float32),
        pltpu.VMEM((SB, D), jnp.float32),
        pltpu.VMEM((SB, D), jnp.float32),
        pltpu.VMEM((SB, D), jnp.float32),
        pltpu.VMEM((SB, 16), jnp.float32),
        pltpu.VMEM((SB, 16), jnp.float32),
        pltpu.VMEM((SB, 16), jnp.float32),
        pltpu.VMEM((SB, 16), jnp.float32),
        pltpu.VMEM(((NPT + 1) * D,), jnp.float32),
        pltpu.VMEM(((NPT + 1) * 16,), jnp.float32),
    ] + [pltpu.SemaphoreType.DMA] * 11
    f = pl.kernel(_sc_l1_body, out_type=out_type, mesh=mesh,
                  scratch_types=scratch, compiler_params=params)
    return f(edges, feats, attr16)


def _sc_layer2(feats, ldstl, srcl, counts):
    mesh = plsc.VectorSubcoreMesh(core_axis_name="c", subcore_axis_name="s")
    params = pltpu.CompilerParams(needs_layout_passes=False,
                                  use_tc_tiling_on_sc=False)
    out_type = jax.ShapeDtypeStruct((NPAD * D,), jnp.float32)
    scratch = [
        pltpu.VMEM((CH + 16,), jnp.int32),
        pltpu.VMEM((CH + 16,), jnp.int32),
        pltpu.VMEM((CH,), jnp.int32),
        pltpu.VMEM((CH,), jnp.int32),
        pltpu.VMEM((SB, D), jnp.float32),
        pltpu.VMEM((SB, D), jnp.float32),
        pltpu.VMEM((SB, D), jnp.float32),
        pltpu.VMEM((SB, D), jnp.float32),
        pltpu.VMEM(((NPT + 1) * D,), jnp.float32),
    ] + [pltpu.SemaphoreType.DMA] * 6
    f = pl.kernel(_sc_l2_body, out_type=out_type, mesh=mesh,
                  scratch_types=scratch, compiler_params=params)
    return f(feats, ldstl, srcl, counts)


def _fix(a):
    return jnp.where(jnp.isfinite(a) & (a > NEG), a, 0.0)


def _dense1_body(ax_ref, aa_ref, wx_ref, wa_ref, b_ref, g_ref, bt_ref, o_ref):
    ax = _fix(ax_ref[...])
    aa = _fix(aa_ref[...])
    h = (jnp.dot(ax, wx_ref[...], preferred_element_type=jnp.float32)
         + jnp.dot(aa, wa_ref[...], preferred_element_type=jnp.float32)
         + b_ref[...])
    row = lax.broadcasted_iota(jnp.int32, (NPAD, 1), 0)
    mask = (row < N).astype(jnp.float32)
    mean = jnp.sum(h * mask, axis=0, keepdims=True) * (1.0 / N)
    d = (h - mean) * mask
    var = jnp.sum(d * d, axis=0, keepdims=True) * (1.0 / N)
    hn = (h - mean) * lax.rsqrt(var + EPS) * g_ref[...] + bt_ref[...]
    o_ref[...] = jnp.maximum(hn, 0.0) * mask


def _dense2_body(ax_ref, aa_ref, wx_ref, wa_ref, b_ref, g_ref, bt_ref,
                 wout_ref, bout_ref, batch_ref, o_ref):
    ax = _fix(ax_ref[...])
    aa = _fix(aa_ref[...])
    h = (jnp.dot(ax, wx_ref[...], preferred_element_type=jnp.float32)
         + jnp.dot(aa, wa_ref[...], preferred_element_type=jnp.float32)
         + b_ref[...])
    row = lax.broadcasted_iota(jnp.int32, (NPAD, 1), 0)
    mask = (row < N).astype(jnp.float32)
    mean = jnp.sum(h * mask, axis=0, keepdims=True) * (1.0 / N)
    d = (h - mean) * mask
    var = jnp.sum(d * d, axis=0, keepdims=True) * (1.0 / N)
    hn = (h - mean) * lax.rsqrt(var + EPS) * g_ref[...] + bt_ref[...]
    h1 = jnp.maximum(hn, 0.0)
    valid = row < N
    b = batch_ref[...]
    cols = []
    for gidx in range(G):
        sel = jnp.where((b == gidx) & valid, h1, -jnp.inf)
        cols.append(jnp.max(sel, axis=0, keepdims=True))
    pooled = jnp.concatenate(cols, axis=0)
    pooled = jnp.where(jnp.isfinite(pooled), pooled, 0.0)
    o_ref[...] = (jnp.dot(pooled, wout_ref[...],
                          preferred_element_type=jnp.float32) + bout_ref[...])


def kernel(x, edge_index, edge_attr, batch, W0, b0, g0, bt0, W1, b1, g1, bt1,
           Wout, bout):
    dst = edge_index[1].reshape(NCHUNK, 1, CH)
    src = edge_index[0].reshape(NCHUNK, 1, CH)
    edges = jnp.concatenate([dst, src], axis=1).reshape(2 * NCHUNK, CH)
    attr16 = jnp.pad(edge_attr, ((0, 0), (0, 10)))
    xpad = jnp.pad(x, ((0, NPAD - N), (0, 0)))
    batch2d = jnp.pad(batch, (0, NPAD - N), constant_values=G).reshape(NPAD, 1)

    aggx0_f, aggattr_f, ldstl, srcl, counts = _sc_layer1(xpad, edges, attr16)
    aggx0 = aggx0_f.reshape(NPAD, D)
    aggattr = aggattr_f.reshape(NPAD, 16)

    W0x = W0[:D]
    W0a = jnp.pad(W0[D:], ((0, 10), (0, 0)))
    h0 = pl.pallas_call(
        _dense1_body,
        out_shape=jax.ShapeDtypeStruct((NPAD, H), jnp.float32),
    )(aggx0, aggattr, W0x, W0a, b0.reshape(1, H), g0.reshape(1, H),
      bt0.reshape(1, H))

    aggx1 = _sc_layer2(h0, ldstl, srcl, counts).reshape(NPAD, D)

    W1x = W1[:H]
    W1a = jnp.pad(W1[H:], ((0, 10), (0, 0)))
    out = pl.pallas_call(
        _dense2_body,
        out_shape=jax.ShapeDtypeStruct((G, 1), jnp.float32),
    )(aggx1, aggattr, W1x, W1a, b1.reshape(1, H), g1.reshape(1, H),
      bt1.reshape(1, H), Wout, bout.reshape(1, 1), batch2d)
    return out
